# baseline (speedup 1.0000x reference)
import functools

import jax
import jax.numpy as jnp
from jax import lax
from jax.experimental import pallas as pl
from jax.experimental.pallas import tpu as pltpu
from jax.experimental.pallas import tpu_sc as plsc

N_PAD = 10112
D = 128
NREL = 8
NC, NS, L = 2, 16, 16
CHUNK = 128
ROWS_PER_TILE = N_PAD // NS


def _matmul_body(x_ref, w_ref, y_ref):
    y_ref[0] = jnp.dot(x_ref[...], w_ref[0], preferred_element_type=jnp.float32)


MM_BLK = N_PAD // 8


def _matmul(x_pad, w_all):
    return pl.pallas_call(
        _matmul_body,
        grid=(8, w_all.shape[0]),
        in_specs=[
            pl.BlockSpec((MM_BLK, D), lambda i, r: (i, 0)),
            pl.BlockSpec((1, D, D), lambda i, r: (r, 0, 0)),
        ],
        out_specs=pl.BlockSpec((1, MM_BLK, D), lambda i, r: (r, i, 0)),
        out_shape=jax.ShapeDtypeStruct((w_all.shape[0], N_PAD, D), jnp.float32),
    )(x_pad, w_all)


NBUF = 1
C0 = 80
ALLOC_CHUNKS = NS * C0
SLAB = 8


def _sc_body(table_hbm, gidx_hbm, dst_hbm, out_hbm,
             gidx_v, dst_v, rows0, rows1, acc_sh, sem0, sem1):
    rows_b = (rows0, rows1)
    sem_b = (sem0, sem1)
    cid = lax.axis_index("c")
    sid = lax.axis_index("s")
    base = pl.multiple_of(sid * C0, 8)

    @pl.when(cid == 0)
    def _():
        init_base = NREL * N_PAD + sid * ROWS_PER_TILE
        pltpu.sync_copy(table_hbm.at[pl.ds(init_base, ROWS_PER_TILE)],
                        acc_sh.at[pl.ds(sid * ROWS_PER_TILE, ROWS_PER_TILE)])

        def _load(i, _):
            sl_h = pl.ds(base + i * SLAB, SLAB)
            sl_v = pl.ds(i * SLAB, SLAB)
            pltpu.sync_copy(gidx_hbm.at[sl_h], gidx_v.at[sl_v])
            pltpu.sync_copy(dst_hbm.at[sl_h], dst_v.at[sl_v])
            return 0
        lax.fori_loop(0, C0 // SLAB, _load, 0)

    plsc.subcore_barrier()

    def _grp(g, _):
        descs = []
        for b in range(NBUF):
            c = NBUF * g + b
            descs.append(pltpu.async_copy(
                table_hbm.at[gidx_v.at[c]], rows_b[b], sem_b[b]))
        for b in range(NBUF):
            c = NBUF * g + b
            descs[b].wait()
            pltpu.sync_copy(rows_b[b], acc_sh.at[dst_v.at[c]], add=True)
        return 0
    nch = jnp.where(cid == 0, C0, 0)
    lax.fori_loop(0, nch // NBUF, _grp, 0)

    plsc.subcore_barrier()

    @pl.when(cid == 0)
    def _():
        pltpu.sync_copy(acc_sh.at[pl.ds(sid * ROWS_PER_TILE, ROWS_PER_TILE)],
                        out_hbm.at[pl.ds(sid * ROWS_PER_TILE, ROWS_PER_TILE)])


_sc_scatter = functools.partial(
    pl.kernel,
    out_type=jax.ShapeDtypeStruct((N_PAD, D), jnp.float32),
    mesh=plsc.VectorSubcoreMesh(core_axis_name="c", subcore_axis_name="s",
                                num_cores=NC, num_subcores=NS),
    scratch_types=[
        pltpu.VMEM((C0, CHUNK), jnp.int32),
        pltpu.VMEM((C0, CHUNK), jnp.int32),
        pltpu.VMEM((CHUNK, D), jnp.float32),
        pltpu.VMEM((CHUNK, D), jnp.float32),
        pltpu.VMEM_SHARED((N_PAD, D), jnp.float32),
        pltpu.SemaphoreType.DMA,
        pltpu.SemaphoreType.DMA,
    ],
)(_sc_body)


def _combine_body(p_ref, o_ref):
    o_ref[...] = jnp.maximum(p_ref[...], 0.0)


def _combine(partial, n):
    blk = n // 5
    return pl.pallas_call(
        _combine_body,
        grid=(5,),
        in_specs=[pl.BlockSpec((blk, D), lambda i: (i, 0))],
        out_specs=pl.BlockSpec((blk, D), lambda i: (i, 0)),
        out_shape=jax.ShapeDtypeStruct((n, D), jnp.float32),
    )(partial)


def kernel(x, weight, self_loop_w, edge_index, edge_type):
    n = x.shape[0]
    ne = edge_type.shape[0]
    x_pad = jnp.pad(x, ((0, N_PAD - n), (0, 0)))
    w_all = jnp.concatenate([weight, self_loop_w.T[None]], axis=0)
    table = _matmul(x_pad, w_all).reshape(w_all.shape[0] * N_PAD, D)

    pad = ALLOC_CHUNKS * CHUNK - ne
    gidx = edge_type * N_PAD + edge_index[0]
    gidx_p = jnp.pad(gidx, (0, pad)).reshape(ALLOC_CHUNKS, CHUNK)
    dst_p = jnp.pad(edge_index[1], (0, pad),
                    constant_values=n).reshape(ALLOC_CHUNKS, CHUNK)

    partial = _sc_scatter(table, gidx_p, dst_p)
    return _combine(partial, n)


# speedup vs baseline: 1.1836x; 1.1836x over previous
import functools

import jax
import jax.numpy as jnp
from jax import lax
from jax.experimental import pallas as pl
from jax.experimental.pallas import tpu as pltpu
from jax.experimental.pallas import tpu_sc as plsc

N_PAD = 10112
D = 128
NREL = 8
NC, NS, L = 2, 16, 16
CHUNK = 128
ROWS_PER_TILE = N_PAD // NS


def _matmul_body(x_ref, w_ref, y_ref):
    y_ref[0] = jnp.dot(x_ref[...], w_ref[0], preferred_element_type=jnp.float32)


MM_BLK = N_PAD // 8


def _matmul(x_pad, w_all):
    return pl.pallas_call(
        _matmul_body,
        grid=(8, w_all.shape[0]),
        in_specs=[
            pl.BlockSpec((MM_BLK, D), lambda i, r: (i, 0)),
            pl.BlockSpec((1, D, D), lambda i, r: (r, 0, 0)),
        ],
        out_specs=pl.BlockSpec((1, MM_BLK, D), lambda i, r: (r, i, 0)),
        out_shape=jax.ShapeDtypeStruct((w_all.shape[0], N_PAD, D), jnp.float32),
    )(x_pad, w_all)


C0 = 80
ALLOC_CHUNKS = NS * C0
SG = 8
NSG = C0 // SG


def _sc_body(table_hbm, gidx_hbm, dst_hbm, out_hbm,
             gidx_s, dst_s, rows0, rows1, acc_sh, sem0, sem1):
    rows_b = (rows0, rows1)
    sem_b = (sem0, sem1)
    cid = lax.axis_index("c")
    sid = lax.axis_index("s")
    base = pl.multiple_of(sid * C0, 8)

    def _slab(slot, sg):
        sl_h = pl.ds(base + sg * SG, SG)
        pltpu.sync_copy(gidx_hbm.at[sl_h], gidx_s.at[slot])
        pltpu.sync_copy(dst_hbm.at[sl_h], dst_s.at[slot])

    @pl.when(cid == 0)
    def _():
        init_base = NREL * N_PAD + sid * ROWS_PER_TILE
        pltpu.sync_copy(table_hbm.at[pl.ds(init_base, ROWS_PER_TILE)],
                        acc_sh.at[pl.ds(sid * ROWS_PER_TILE, ROWS_PER_TILE)])
        _slab(0, 0)

    plsc.subcore_barrier()

    def _sg_body(p, sg):
        d = [pltpu.async_copy(table_hbm.at[gidx_s.at[p, 0]], rows0, sem0),
             pltpu.async_copy(table_hbm.at[gidx_s.at[p, 1]], rows1, sem1)]

        @pl.when(sg + 1 < NSG)
        def _():
            _slab(1 - p, sg + 1)

        for k in range(SG):
            b = k % 2
            d[b].wait()
            pltpu.sync_copy(rows_b[b], acc_sh.at[dst_s.at[p, k]], add=True)
            if k + 2 < SG:
                d[b] = pltpu.async_copy(
                    table_hbm.at[gidx_s.at[p, k + 2]], rows_b[b], sem_b[b])

    def _outer(g2, _):
        for p in (0, 1):
            _sg_body(p, 2 * g2 + p)
        return 0
    lax.fori_loop(0, jnp.where(cid == 0, NSG // 2, 0), _outer, 0)

    plsc.subcore_barrier()

    @pl.when(cid == 0)
    def _():
        pltpu.sync_copy(acc_sh.at[pl.ds(sid * ROWS_PER_TILE, ROWS_PER_TILE)],
                        out_hbm.at[pl.ds(sid * ROWS_PER_TILE, ROWS_PER_TILE)])


_sc_scatter = functools.partial(
    pl.kernel,
    out_type=jax.ShapeDtypeStruct((N_PAD, D), jnp.float32),
    mesh=plsc.VectorSubcoreMesh(core_axis_name="c", subcore_axis_name="s",
                                num_cores=NC, num_subcores=NS),
    scratch_types=[
        pltpu.VMEM((2, SG, CHUNK), jnp.int32),
        pltpu.VMEM((2, SG, CHUNK), jnp.int32),
        pltpu.VMEM((CHUNK, D), jnp.float32),
        pltpu.VMEM((CHUNK, D), jnp.float32),
        pltpu.VMEM_SHARED((N_PAD, D), jnp.float32),
        pltpu.SemaphoreType.DMA,
        pltpu.SemaphoreType.DMA,
    ],
)(_sc_body)


def _combine_body(p_ref, o_ref):
    o_ref[...] = jnp.maximum(p_ref[...], 0.0)


def _combine(partial, n):
    blk = n // 5
    return pl.pallas_call(
        _combine_body,
        grid=(5,),
        in_specs=[pl.BlockSpec((blk, D), lambda i: (i, 0))],
        out_specs=pl.BlockSpec((blk, D), lambda i: (i, 0)),
        out_shape=jax.ShapeDtypeStruct((n, D), jnp.float32),
    )(partial)


def kernel(x, weight, self_loop_w, edge_index, edge_type):
    n = x.shape[0]
    ne = edge_type.shape[0]
    x_pad = jnp.pad(x, ((0, N_PAD - n), (0, 0)))
    w_all = jnp.concatenate([weight, self_loop_w.T[None]], axis=0)
    table = _matmul(x_pad, w_all).reshape(w_all.shape[0] * N_PAD, D)

    pad = ALLOC_CHUNKS * CHUNK - ne
    gidx = edge_type * N_PAD + edge_index[0]
    gidx_p = jnp.pad(gidx, (0, pad)).reshape(ALLOC_CHUNKS, CHUNK)
    dst_p = jnp.pad(edge_index[1], (0, pad),
                    constant_values=n).reshape(ALLOC_CHUNKS, CHUNK)

    partial = _sc_scatter(table, gidx_p, dst_p)
    return _combine(partial, n)


# speedup vs baseline: 1.1896x; 1.0050x over previous
import functools

import jax
import jax.numpy as jnp
from jax import lax
from jax.experimental import pallas as pl
from jax.experimental.pallas import tpu as pltpu
from jax.experimental.pallas import tpu_sc as plsc

N_PAD = 10112
D = 128
NREL = 8
NC, NS, L = 2, 16, 16
CHUNK = 128
ROWS_PER_TILE = N_PAD // NS


def _matmul_body(x_ref, w_ref, y_ref):
    y_ref[0] = jnp.dot(x_ref[...], w_ref[0], preferred_element_type=jnp.float32)


MM_BLK = N_PAD // 8


def _matmul(x_pad, w_all):
    return pl.pallas_call(
        _matmul_body,
        grid=(8, w_all.shape[0]),
        in_specs=[
            pl.BlockSpec((MM_BLK, D), lambda i, r: (i, 0)),
            pl.BlockSpec((1, D, D), lambda i, r: (r, 0, 0)),
        ],
        out_specs=pl.BlockSpec((1, MM_BLK, D), lambda i, r: (r, i, 0)),
        out_shape=jax.ShapeDtypeStruct((w_all.shape[0], N_PAD, D), jnp.float32),
    )(x_pad, w_all)


C0 = 80
ALLOC_CHUNKS = NS * C0
SG = 8
NSG = C0 // SG


def _sc_body(table_hbm, gidx_hbm, dst_hbm, out_hbm,
             gidx_s, dst_s, rows0, rows1, acc_sh, sem0, sem1, sem_i):
    rows_b = (rows0, rows1)
    sem_b = (sem0, sem1)
    cid = lax.axis_index("c")
    sid = lax.axis_index("s")
    base = pl.multiple_of(sid * C0, 8)

    def _slab_start(slot, sg):
        sl_h = pl.ds(base + sg * SG, SG)
        pltpu.async_copy(gidx_hbm.at[sl_h], gidx_s.at[slot], sem_i)
        pltpu.async_copy(dst_hbm.at[sl_h], dst_s.at[slot], sem_i)

    def _slab_wait(slot):
        pltpu.make_async_copy(gidx_hbm.at[pl.ds(0, SG)], gidx_s.at[slot],
                              sem_i).wait()
        pltpu.make_async_copy(dst_hbm.at[pl.ds(0, SG)], dst_s.at[slot],
                              sem_i).wait()

    @pl.when(cid == 0)
    def _():
        _slab_start(0, 0)
        init_base = NREL * N_PAD + sid * ROWS_PER_TILE
        pltpu.sync_copy(table_hbm.at[pl.ds(init_base, ROWS_PER_TILE)],
                        acc_sh.at[pl.ds(sid * ROWS_PER_TILE, ROWS_PER_TILE)])

    plsc.subcore_barrier()

    def _sg_body(p, sg):
        _slab_wait(p)
        d = [pltpu.async_copy(table_hbm.at[gidx_s.at[p, 0]], rows0, sem0),
             pltpu.async_copy(table_hbm.at[gidx_s.at[p, 1]], rows1, sem1)]

        @pl.when(sg + 1 < NSG)
        def _():
            _slab_start(1 - p, sg + 1)

        for k in range(SG):
            b = k % 2
            d[b].wait()
            pltpu.sync_copy(rows_b[b], acc_sh.at[dst_s.at[p, k]], add=True)
            if k + 2 < SG:
                d[b] = pltpu.async_copy(
                    table_hbm.at[gidx_s.at[p, k + 2]], rows_b[b], sem_b[b])

    def _outer(g2, _):
        for p in (0, 1):
            _sg_body(p, 2 * g2 + p)
        return 0
    lax.fori_loop(0, jnp.where(cid == 0, NSG // 2, 0), _outer, 0)

    plsc.subcore_barrier()

    @pl.when(cid == 0)
    def _():
        pltpu.sync_copy(acc_sh.at[pl.ds(sid * ROWS_PER_TILE, ROWS_PER_TILE)],
                        out_hbm.at[pl.ds(sid * ROWS_PER_TILE, ROWS_PER_TILE)])


_sc_scatter = functools.partial(
    pl.kernel,
    out_type=jax.ShapeDtypeStruct((N_PAD, D), jnp.float32),
    mesh=plsc.VectorSubcoreMesh(core_axis_name="c", subcore_axis_name="s",
                                num_cores=NC, num_subcores=NS),
    scratch_types=[
        pltpu.VMEM((2, SG, CHUNK), jnp.int32),
        pltpu.VMEM((2, SG, CHUNK), jnp.int32),
        pltpu.VMEM((CHUNK, D), jnp.float32),
        pltpu.VMEM((CHUNK, D), jnp.float32),
        pltpu.VMEM_SHARED((N_PAD, D), jnp.float32),
        pltpu.SemaphoreType.DMA,
        pltpu.SemaphoreType.DMA,
        pltpu.SemaphoreType.DMA,
    ],
)(_sc_body)


def _combine_body(p_ref, o_ref):
    o_ref[...] = jnp.maximum(p_ref[...], 0.0)


def _combine(partial, n):
    blk = n // 5
    return pl.pallas_call(
        _combine_body,
        grid=(5,),
        in_specs=[pl.BlockSpec((blk, D), lambda i: (i, 0))],
        out_specs=pl.BlockSpec((blk, D), lambda i: (i, 0)),
        out_shape=jax.ShapeDtypeStruct((n, D), jnp.float32),
    )(partial)


def kernel(x, weight, self_loop_w, edge_index, edge_type):
    n = x.shape[0]
    ne = edge_type.shape[0]
    x_pad = jnp.pad(x, ((0, N_PAD - n), (0, 0)))
    w_all = jnp.concatenate([weight, self_loop_w.T[None]], axis=0)
    table = _matmul(x_pad, w_all).reshape(w_all.shape[0] * N_PAD, D)

    pad = ALLOC_CHUNKS * CHUNK - ne
    gidx = edge_type * N_PAD + edge_index[0]
    gidx_p = jnp.pad(gidx, (0, pad)).reshape(ALLOC_CHUNKS, CHUNK)
    dst_p = jnp.pad(edge_index[1], (0, pad),
                    constant_values=n).reshape(ALLOC_CHUNKS, CHUNK)

    partial = _sc_scatter(table, gidx_p, dst_p)
    return _combine(partial, n)


# speedup vs baseline: 1.2622x; 1.0610x over previous
import functools

import jax
import jax.numpy as jnp
from jax import lax
from jax.experimental import pallas as pl
from jax.experimental.pallas import tpu as pltpu
from jax.experimental.pallas import tpu_sc as plsc

N_PAD = 10112
D = 128
NREL = 8
NC, NS, L = 2, 16, 16
CHUNK = 128
ROWS_PER_TILE = N_PAD // NS


def _matmul_body(x_ref, w_ref, y_ref):
    y_ref[0] = jnp.dot(x_ref[...], w_ref[0], preferred_element_type=jnp.float32)


MM_BLK = N_PAD // 8


def _matmul(x_pad, w_all):
    return pl.pallas_call(
        _matmul_body,
        grid=(8, w_all.shape[0]),
        in_specs=[
            pl.BlockSpec((MM_BLK, D), lambda i, r: (i, 0)),
            pl.BlockSpec((1, D, D), lambda i, r: (r, 0, 0)),
        ],
        out_specs=pl.BlockSpec((1, MM_BLK, D), lambda i, r: (r, i, 0)),
        out_shape=jax.ShapeDtypeStruct((w_all.shape[0], N_PAD, D), jnp.float32),
    )(x_pad, w_all)


NBUF = 2
C0 = 64
C1 = 16
ALLOC_CHUNKS = NS * C0 + (NS - 1) * C1 + C0


def _sc_body(table_hbm, zero_hbm, gidx_hbm, dst_hbm, out_hbm,
             gidx_v, dst_v, rows0, rows1, acc_sh, sem0, sem1):
    rows_b = (rows0, rows1)
    sem_b = (sem0, sem1)
    cid = lax.axis_index("c")
    sid = lax.axis_index("s")
    rows_sl = pl.ds(sid * ROWS_PER_TILE, ROWS_PER_TILE)

    @pl.when(cid == 0)
    def _():
        init_base = NREL * N_PAD + sid * ROWS_PER_TILE
        pltpu.sync_copy(table_hbm.at[pl.ds(init_base, ROWS_PER_TILE)],
                        acc_sh.at[rows_sl])

    @pl.when(cid == 1)
    def _():
        pltpu.sync_copy(zero_hbm.at[rows_sl], acc_sh.at[rows_sl])

    plsc.subcore_barrier()

    nch = jnp.where(cid == 0, C0, C1)
    base = pl.multiple_of(jnp.where(cid == 0, sid * C0, NS * C0 + sid * C1), 8)

    pltpu.sync_copy(gidx_hbm.at[pl.ds(base, C0)], gidx_v)
    pltpu.sync_copy(dst_hbm.at[pl.ds(base, C0)], dst_v)

    def _grp(g, _):
        descs = []
        for b in range(NBUF):
            c = NBUF * g + b
            descs.append(pltpu.async_copy(
                table_hbm.at[gidx_v.at[c]], rows_b[b], sem_b[b]))
        for b in range(NBUF):
            c = NBUF * g + b
            descs[b].wait()
            pltpu.sync_copy(rows_b[b], acc_sh.at[dst_v.at[c]], add=True)
        return 0
    lax.fori_loop(0, nch // NBUF, _grp, 0)

    plsc.subcore_barrier()
    pltpu.sync_copy(acc_sh.at[rows_sl], out_hbm.at[cid, rows_sl])


_sc_scatter = functools.partial(
    pl.kernel,
    out_type=jax.ShapeDtypeStruct((NC, N_PAD, D), jnp.float32),
    mesh=plsc.VectorSubcoreMesh(core_axis_name="c", subcore_axis_name="s",
                                num_cores=NC, num_subcores=NS),
    scratch_types=[
        pltpu.VMEM((C0, CHUNK), jnp.int32),
        pltpu.VMEM((C0, CHUNK), jnp.int32),
        pltpu.VMEM((CHUNK, D), jnp.float32),
        pltpu.VMEM((CHUNK, D), jnp.float32),
        pltpu.VMEM_SHARED((N_PAD, D), jnp.float32),
        pltpu.SemaphoreType.DMA,
        pltpu.SemaphoreType.DMA,
    ],
)(_sc_body)


def _combine_body(p_ref, o_ref):
    o_ref[...] = jnp.maximum(p_ref[0] + p_ref[1], 0.0)


def _combine(partials, n):
    blk = n // 5
    return pl.pallas_call(
        _combine_body,
        grid=(5,),
        in_specs=[pl.BlockSpec((NC, blk, D), lambda i: (0, i, 0))],
        out_specs=pl.BlockSpec((blk, D), lambda i: (i, 0)),
        out_shape=jax.ShapeDtypeStruct((n, D), jnp.float32),
    )(partials)


def kernel(x, weight, self_loop_w, edge_index, edge_type):
    n = x.shape[0]
    ne = edge_type.shape[0]
    x_pad = jnp.pad(x, ((0, N_PAD - n), (0, 0)))
    w_all = jnp.concatenate([weight, self_loop_w.T[None]], axis=0)
    table = _matmul(x_pad, w_all).reshape(w_all.shape[0] * N_PAD, D)

    pad = ALLOC_CHUNKS * CHUNK - ne
    gidx = edge_type * N_PAD + edge_index[0]
    gidx_p = jnp.pad(gidx, (0, pad)).reshape(ALLOC_CHUNKS, CHUNK)
    dst_p = jnp.pad(edge_index[1], (0, pad),
                    constant_values=n).reshape(ALLOC_CHUNKS, CHUNK)

    zero = jnp.zeros((N_PAD, D), jnp.float32)
    partials = _sc_scatter(table, zero, gidx_p, dst_p)
    return _combine(partials, n)
